# baseline (speedup 1.0000x reference)
import functools

import jax
import jax.numpy as jnp
from jax import lax
from jax.experimental import pallas as pl
from jax.experimental.pallas import tpu as pltpu
from jax.experimental.pallas import tpu_sc as plsc

B, L, BB, D, V = 4096, 50, 100, 128, 100000
N = B * L
NC, NS = 2, 16
NW = NC * NS
SLICES = (512, 1024, 1024, 1536)
OFFS = (0, 512, 1536, 2560)
K = len(SLICES)
C = 80


def _sc_gather(bs, emb_table, idx3):
    nsl = bs * L
    rpw = nsl // NW
    nch = rpw // C
    assert nch % 2 == 0 and rpw % C == 0
    mesh = plsc.VectorSubcoreMesh(core_axis_name="c", subcore_axis_name="s")

    @functools.partial(
        pl.kernel,
        out_type=jax.ShapeDtypeStruct((nsl, D), jnp.float32),
        mesh=mesh,
        scratch_types=[
            pltpu.VMEM((nch, C), jnp.int32),
            pltpu.VMEM((C, D), jnp.float32),
            pltpu.VMEM((C, D), jnp.float32),
            pltpu.SemaphoreType.DMA,
            pltpu.SemaphoreType.DMA,
        ],
    )
    def k(table_hbm, idx_hbm, out_hbm, idx_v, rows0, rows1, g0, g1):
        wid = lax.axis_index("s") * NC + lax.axis_index("c")
        base = wid * rpw
        pltpu.sync_copy(idx_hbm.at[wid], idx_v)
        pltpu.async_copy(table_hbm.at[idx_v.at[0]], rows0, g0)

        def pair(jj, carry):
            j0 = 2 * jj
            pltpu.async_copy(table_hbm.at[idx_v.at[j0 + 1]], rows1, g1)
            pltpu.make_async_copy(table_hbm.at[idx_v.at[j0]], rows0, g0).wait()
            pltpu.sync_copy(rows0, out_hbm.at[pl.ds(base + j0 * C, C)])

            @pl.when(jj + 1 < nch // 2)
            def _():
                pltpu.async_copy(table_hbm.at[idx_v.at[j0 + 2]], rows0, g0)

            pltpu.make_async_copy(
                table_hbm.at[idx_v.at[j0 + 1]], rows1, g1).wait()
            pltpu.sync_copy(rows1, out_hbm.at[pl.ds(base + (j0 + 1) * C, C)])
            return carry

        lax.fori_loop(0, nch // 2, pair, 0)

    return k(emb_table, idx3)


G = 256
R = G * L


def _tc_body(conc_ref, mm_ref, pm_ref, w1_ref, w2_ref, al_ref, wl_ref,
             xm_ref, cls_ref, pade_ref, maske_ref, *rest):
    (out_ref,) = rest[-1:]
    c2 = conc_ref[...]
    c2 = jnp.where(jnp.isnan(c2), jnp.float32(0.0), c2)
    xb = lax.broadcast_in_dim(c2, (L, G, BB), (0, 1))
    v1 = xb * w1_ref[...][None]
    v1 = jnp.where(v1 >= 0, v1, 0.01 * v1)
    v1f = v1.reshape(L * G, BB)
    v2 = lax.dot_general(v1f, w2_ref[...], (((1,), (1,)), ((), ())),
                         preferred_element_type=jnp.float32)
    v2 = v2 + al_ref[...] * v1f
    e = jnp.exp(v2)
    v3 = e / jnp.sum(e, axis=-1, keepdims=True)
    xc = lax.dot_general(v3, wl_ref[...], (((1,), (1,)), ((), ())),
                         preferred_element_type=jnp.float32)
    xc3 = xc.reshape(L, G, D)
    mm3 = lax.broadcast_in_dim(mm_ref[...], (L, G, D), (0, 1))
    xc3 = jnp.where(mm3 == 1, maske_ref[...][None], xc3)
    merged = (xm_ref[...] + xc3) * 0.5
    pm3 = lax.broadcast_in_dim(pm_ref[...], (L, G, D), (0, 1))
    merged = jnp.where(pm3 == 1, pade_ref[...][None], merged)
    cls_tile = jnp.broadcast_to(cls_ref[...][None], (1, G, D))
    out_ref[...] = jnp.concatenate([cls_tile, merged], axis=0)


def _tc_combine(bs, boff, concT, mmT, pmT, w1t, w2, al2, wl, xm3,
                cls2, pade2, maske2, prev):
    off = boff // G
    in_specs = [
        pl.BlockSpec((L, G), lambda i: (0, i + off)),
        pl.BlockSpec((L, G), lambda i: (0, i + off)),
        pl.BlockSpec((L, G), lambda i: (0, i + off)),
        pl.BlockSpec((1, BB), lambda i: (0, 0)),
        pl.BlockSpec((BB, BB), lambda i: (0, 0)),
        pl.BlockSpec((1, BB), lambda i: (0, 0)),
        pl.BlockSpec((D, BB), lambda i: (0, 0)),
        pl.BlockSpec((L, G, D), lambda i: (0, i, 0)),
        pl.BlockSpec((1, D), lambda i: (0, 0)),
        pl.BlockSpec((1, D), lambda i: (0, 0)),
        pl.BlockSpec((1, D), lambda i: (0, 0)),
    ]
    args = [concT, mmT, pmT, w1t, w2, al2, wl, xm3, cls2, pade2, maske2]
    aliases = {}
    if prev is not None:
        in_specs.append(pl.BlockSpec(memory_space=pl.ANY))
        args.append(prev)
        aliases = {11: 0}
    return pl.pallas_call(
        _tc_body,
        grid=(bs // G,),
        in_specs=in_specs,
        out_specs=pl.BlockSpec((L + 1, G, D), lambda i: (0, i + off, 0)),
        out_shape=jax.ShapeDtypeStruct((L + 1, B, D), jnp.float32),
        input_output_aliases=aliases,
    )(*args)


def kernel(concentration, identifier, masking_mask, padding_mask,
           W1, W2, alpha, W_lookup, emb_table, cls_emb, pad_emb, mask_emb):
    ident = identifier.astype(jnp.int32)
    idxs = [ident[o:o + bs].T.reshape(NW, bs * L // (NW * C), C)
            for bs, o in zip(SLICES, OFFS)]
    concT = concentration.T
    mmT = masking_mask.astype(jnp.int32).T
    pmT = padding_mask.astype(jnp.int32).T
    w1t = W1.reshape(1, BB)
    al2 = alpha.reshape(1, BB)
    cls2 = cls_emb.reshape(1, D)
    pade2 = pad_emb.reshape(1, D)
    maske2 = mask_emb.reshape(1, D)

    xms = [_sc_gather(bs, emb_table, idxs[k]).reshape(L, bs, D)
           for k, bs in enumerate(SLICES)]
    out = None
    for k, (bs, o) in enumerate(zip(SLICES, OFFS)):
        out = _tc_combine(bs, o, concT, mmT, pmT, w1t, W2, al2, W_lookup,
                          xms[k], cls2, pade2, maske2, out)
    return out.transpose(1, 0, 2)


# speedup vs baseline: 1.0124x; 1.0124x over previous
import functools

import jax
import jax.numpy as jnp
from jax import lax
from jax.experimental import pallas as pl
from jax.experimental.pallas import tpu as pltpu
from jax.experimental.pallas import tpu_sc as plsc

B, L, BB, D, V = 4096, 50, 100, 128, 100000
N = B * L
NC, NS = 2, 16
NW = NC * NS
SLICES = (1024, 1024, 1024, 1024)
OFFS = (0, 1024, 2048, 3072)
K = len(SLICES)
C = 80


def _sc_gather(bs, emb_table, idx3):
    nsl = bs * L
    rpw = nsl // NW
    nch = rpw // C
    assert nch % 2 == 0 and rpw % C == 0
    mesh = plsc.VectorSubcoreMesh(core_axis_name="c", subcore_axis_name="s")

    @functools.partial(
        pl.kernel,
        out_type=jax.ShapeDtypeStruct((nsl, D), jnp.float32),
        mesh=mesh,
        scratch_types=[
            pltpu.VMEM((nch, C), jnp.int32),
            pltpu.VMEM((C, D), jnp.float32),
            pltpu.VMEM((C, D), jnp.float32),
            pltpu.SemaphoreType.DMA,
            pltpu.SemaphoreType.DMA,
        ],
    )
    def k(table_hbm, idx_hbm, out_hbm, idx_v, rows0, rows1, g0, g1):
        wid = lax.axis_index("s") * NC + lax.axis_index("c")
        base = wid * rpw
        pltpu.sync_copy(idx_hbm.at[wid], idx_v)
        pltpu.async_copy(table_hbm.at[idx_v.at[0]], rows0, g0)

        def pair(jj, carry):
            j0 = 2 * jj
            pltpu.async_copy(table_hbm.at[idx_v.at[j0 + 1]], rows1, g1)
            pltpu.make_async_copy(table_hbm.at[idx_v.at[j0]], rows0, g0).wait()
            pltpu.sync_copy(rows0, out_hbm.at[pl.ds(base + j0 * C, C)])

            @pl.when(jj + 1 < nch // 2)
            def _():
                pltpu.async_copy(table_hbm.at[idx_v.at[j0 + 2]], rows0, g0)

            pltpu.make_async_copy(
                table_hbm.at[idx_v.at[j0 + 1]], rows1, g1).wait()
            pltpu.sync_copy(rows1, out_hbm.at[pl.ds(base + (j0 + 1) * C, C)])
            return carry

        lax.fori_loop(0, nch // 2, pair, 0)

    return k(emb_table, idx3)


G = 256
R = G * L


def _tc_body(conc_ref, mm_ref, pm_ref, w1_ref, w2_ref, al_ref, wl_ref,
             xm_ref, cls_ref, pade_ref, maske_ref, *rest):
    (out_ref,) = rest[-1:]
    c2 = conc_ref[...]
    c2 = jnp.where(jnp.isnan(c2), jnp.float32(0.0), c2)
    xb = lax.broadcast_in_dim(c2, (L, G, BB), (0, 1))
    v1 = xb * w1_ref[...][None]
    v1 = jnp.where(v1 >= 0, v1, 0.01 * v1)
    v1f = v1.reshape(L * G, BB)
    v2 = lax.dot_general(v1f, w2_ref[...], (((1,), (1,)), ((), ())),
                         preferred_element_type=jnp.float32)
    v2 = v2 + al_ref[...] * v1f
    e = jnp.exp(v2)
    v3 = e / jnp.sum(e, axis=-1, keepdims=True)
    xc = lax.dot_general(v3, wl_ref[...], (((1,), (1,)), ((), ())),
                         preferred_element_type=jnp.float32)
    xc3 = xc.reshape(L, G, D)
    mm3 = lax.broadcast_in_dim(mm_ref[...], (L, G, D), (0, 1))
    xc3 = jnp.where(mm3 == 1, maske_ref[...][None], xc3)
    merged = (xm_ref[...] + xc3) * 0.5
    pm3 = lax.broadcast_in_dim(pm_ref[...], (L, G, D), (0, 1))
    merged = jnp.where(pm3 == 1, pade_ref[...][None], merged)
    cls_tile = jnp.broadcast_to(cls_ref[...][None], (1, G, D))
    out_ref[...] = jnp.concatenate([cls_tile, merged], axis=0)


def _tc_combine(bs, boff, concT, mmT, pmT, w1t, w2, al2, wl, xm3,
                cls2, pade2, maske2, prev):
    off = boff // G
    in_specs = [
        pl.BlockSpec((L, G), lambda i: (0, i + off)),
        pl.BlockSpec((L, G), lambda i: (0, i + off)),
        pl.BlockSpec((L, G), lambda i: (0, i + off)),
        pl.BlockSpec((1, BB), lambda i: (0, 0)),
        pl.BlockSpec((BB, BB), lambda i: (0, 0)),
        pl.BlockSpec((1, BB), lambda i: (0, 0)),
        pl.BlockSpec((D, BB), lambda i: (0, 0)),
        pl.BlockSpec((L, G, D), lambda i: (0, i, 0)),
        pl.BlockSpec((1, D), lambda i: (0, 0)),
        pl.BlockSpec((1, D), lambda i: (0, 0)),
        pl.BlockSpec((1, D), lambda i: (0, 0)),
    ]
    args = [concT, mmT, pmT, w1t, w2, al2, wl, xm3, cls2, pade2, maske2]
    aliases = {}
    if prev is not None:
        in_specs.append(pl.BlockSpec(memory_space=pl.ANY))
        args.append(prev)
        aliases = {11: 0}
    return pl.pallas_call(
        _tc_body,
        grid=(bs // G,),
        in_specs=in_specs,
        out_specs=pl.BlockSpec((L + 1, G, D), lambda i: (0, i + off, 0)),
        out_shape=jax.ShapeDtypeStruct((L + 1, B, D), jnp.float32),
        input_output_aliases=aliases,
    )(*args)


def kernel(concentration, identifier, masking_mask, padding_mask,
           W1, W2, alpha, W_lookup, emb_table, cls_emb, pad_emb, mask_emb):
    ident = identifier.astype(jnp.int32)
    idxs = [ident[o:o + bs].T.reshape(NW, bs * L // (NW * C), C)
            for bs, o in zip(SLICES, OFFS)]
    concT = concentration.T
    mmT = masking_mask.astype(jnp.int32).T
    pmT = padding_mask.astype(jnp.int32).T
    w1t = W1.reshape(1, BB)
    al2 = alpha.reshape(1, BB)
    cls2 = cls_emb.reshape(1, D)
    pade2 = pad_emb.reshape(1, D)
    maske2 = mask_emb.reshape(1, D)

    xms = [_sc_gather(bs, emb_table, idxs[k]).reshape(L, bs, D)
           for k, bs in enumerate(SLICES)]
    out = None
    for k, (bs, o) in enumerate(zip(SLICES, OFFS)):
        out = _tc_combine(bs, o, concT, mmT, pmT, w1t, W2, al2, W_lookup,
                          xms[k], cls2, pade2, maske2, out)
    return out.transpose(1, 0, 2)
